# baseline (speedup 1.0000x reference)
import functools

import jax
import jax.numpy as jnp
from jax import lax
from jax.experimental import pallas as pl
from jax.experimental.pallas import tpu as pltpu
from jax.experimental.pallas import tpu_sc as plsc

N = 10000
E = 320000
D = 128
NC = 2
NS = 16
NW = NC * NS
CH = 64
EPT = 10240
NCH = EPT // CH
NBUF = 4
E_PAD = NW * EPT
ROWS_PT = N // NS
DEG_PT = 640
N_DEG = NS * DEG_PT

_mesh = plsc.VectorSubcoreMesh(core_axis_name="c", subcore_axis_name="s")


@functools.partial(
    pl.kernel,
    mesh=_mesh,
    out_type=jax.ShapeDtypeStruct((NC, NS, DEG_PT), jnp.float32),
    scratch_types=[
        pltpu.VMEM((NCH // 4, CH), jnp.int32),
        pltpu.VMEM((NCH // 4, CH), jnp.float32),
        pltpu.VMEM_SHARED((N_DEG,), jnp.float32),
        pltpu.SemaphoreType.DMA,
    ],
)
def _sc_deg(dst_hbm, w_hbm, z_hbm, out_hbm, dst_all, w_all, acc, sem):
    cid = lax.axis_index("c")
    sid = lax.axis_index("s")
    wid = sid * NC + cid
    pltpu.sync_copy(z_hbm, acc.at[pl.ds(sid * DEG_PT, DEG_PT)])
    plsc.subcore_barrier()
    qch = NCH // 4

    def body(q, carry):
        for r in range(4):
            t = q * 4 + r
            pltpu.async_copy(w_all.at[t], acc.at[dst_all.at[t]], sem, add=True)
        for r in range(4):
            t = q * 4 + r
            pltpu.make_async_copy(w_all.at[t], acc.at[dst_all.at[t]], sem).wait()
        return carry

    for h in range(4):
        pltpu.sync_copy(dst_hbm.at[wid, pl.ds(h * qch, qch)], dst_all)
        pltpu.sync_copy(w_hbm.at[wid, pl.ds(h * qch, qch)], w_all)
        lax.fori_loop(0, qch // 4, body, 0)
    plsc.subcore_barrier()
    pltpu.sync_copy(acc.at[pl.ds(sid * DEG_PT, DEG_PT)], out_hbm.at[cid, sid])


@functools.partial(
    pl.kernel,
    mesh=_mesh,
    out_type=jax.ShapeDtypeStruct((NC, NS, ROWS_PT, D), jnp.float32),
    scratch_types=[
        pltpu.VMEM((NCH // 4, CH), jnp.int32),
        pltpu.VMEM((NCH // 4, CH), jnp.int32),
        pltpu.VMEM((NCH // 4, CH), jnp.float32),
        pltpu.VMEM((NBUF, CH, D), jnp.float32),
        pltpu.VMEM_SHARED((N, D), jnp.float32),
    ]
    + [pltpu.SemaphoreType.DMA] * (2 * NBUF),
)
def _sc_main(hs_hbm, src_hbm, dst_hbm, w_hbm, z_hbm, out_hbm,
             src_all, dst_all, w_all, rows, acc, *sems):
    sem_g = sems[:NBUF]
    sem_s = sems[NBUF:]
    cid = lax.axis_index("c")
    sid = lax.axis_index("s")
    wid = sid * NC + cid
    hch = NCH // 4
    pltpu.sync_copy(z_hbm, acc.at[pl.ds(sid * ROWS_PT, ROWS_PT)])
    plsc.subcore_barrier()

    def _slot(t, r):
        pltpu.make_async_copy(hs_hbm.at[src_all.at[t]], rows.at[r], sem_g[r]).wait()

        def scale(g, c2):
            wv16 = w_all[t, pl.ds(g * 16, 16)]
            for k in range(16):
                s = wv16[k]
                e = g * 16 + k
                for j in range(D // 16):
                    sl = pl.ds(j * 16, 16)
                    rows[r, e, sl] = rows[r, e, sl] * s
            return c2

        lax.fori_loop(0, CH // 16, scale, 0)
        pltpu.async_copy(rows.at[r], acc.at[dst_all.at[t]], sem_s[r], add=True)
        r2 = (r + 2) % NBUF

        @pl.when(t >= 2)
        def _():
            pltpu.make_async_copy(rows.at[r2], acc.at[dst_all.at[t - 2]],
                                  sem_s[r2]).wait()

        @pl.when(t + 2 < hch)
        def _():
            pltpu.async_copy(hs_hbm.at[src_all.at[t + 2]], rows.at[r2], sem_g[r2])

    def group(qq, carry):
        for r in range(NBUF):
            _slot(qq * NBUF + r, r)
        return carry

    for h in range(4):
        pltpu.sync_copy(src_hbm.at[wid, pl.ds(h * hch, hch)], src_all)
        pltpu.sync_copy(dst_hbm.at[wid, pl.ds(h * hch, hch)], dst_all)
        pltpu.sync_copy(w_hbm.at[wid, pl.ds(h * hch, hch)], w_all)
        pltpu.async_copy(hs_hbm.at[src_all.at[0]], rows.at[0], sem_g[0])
        pltpu.async_copy(hs_hbm.at[src_all.at[1]], rows.at[1], sem_g[1])
        lax.fori_loop(0, hch // NBUF, group, 0)
        pltpu.make_async_copy(rows.at[(hch - 2) % NBUF],
                              acc.at[dst_all.at[hch - 2]],
                              sem_s[(hch - 2) % NBUF]).wait()
        pltpu.make_async_copy(rows.at[(hch - 1) % NBUF],
                              acc.at[dst_all.at[hch - 1]],
                              sem_s[(hch - 1) % NBUF]).wait()

    plsc.subcore_barrier()
    pltpu.sync_copy(acc.at[pl.ds(sid * ROWS_PT, ROWS_PT)], out_hbm.at[cid, sid])


def _tc_k1_body(text_ref, w_ref, degp_ref, hs_ref, g_ref, dinv_ref):
    h = jnp.dot(text_ref[...], w_ref[...], preferred_element_type=jnp.float32)
    deg = degp_ref[0] + degp_ref[1] + 2.0
    dinv = lax.rsqrt(deg)
    hs_ref[...] = h * dinv
    g_ref[...] = (2.0 * dinv * dinv) * h
    dinv_ref[...] = dinv


def _tc_k2_body(p_ref, g_ref, dinv_ref, out_ref):
    s = dinv_ref[...] * (p_ref[0] + p_ref[1]) + g_ref[...]
    out_ref[...] = jnp.where(s >= 0, s, 0.01 * s)


_BR = 1000


def kernel(text, adj_index, adj_weight, W):
    src = adj_index[0]
    dst = adj_index[1]
    pad = E_PAD - E
    zi = jnp.zeros((pad,), jnp.int32)
    srcp = jnp.concatenate([src, zi]).reshape(NW, NCH, CH)
    dstp = jnp.concatenate([dst, zi]).reshape(NW, NCH, CH)
    wp = jnp.concatenate([adj_weight, jnp.zeros((pad,), jnp.float32)]).reshape(NW, NCH, CH)

    degp = _sc_deg(dstp, wp, jnp.zeros((DEG_PT,), jnp.float32))
    degp3 = degp.reshape(NC, N_DEG, 1)

    nblk = N // _BR
    hs, g, dinv = pl.pallas_call(
        _tc_k1_body,
        grid=(nblk,),
        in_specs=[
            pl.BlockSpec((_BR, D), lambda i: (i, 0)),
            pl.BlockSpec((D, D), lambda i: (0, 0)),
            pl.BlockSpec((NC, _BR, 1), lambda i: (0, i, 0)),
        ],
        out_specs=[
            pl.BlockSpec((_BR, D), lambda i: (i, 0)),
            pl.BlockSpec((_BR, D), lambda i: (i, 0)),
            pl.BlockSpec((_BR, 1), lambda i: (i, 0)),
        ],
        out_shape=[
            jax.ShapeDtypeStruct((N, D), jnp.float32),
            jax.ShapeDtypeStruct((N, D), jnp.float32),
            jax.ShapeDtypeStruct((N, 1), jnp.float32),
        ],
    )(text, W, degp3)

    P = _sc_main(hs, srcp, dstp, wp, jnp.zeros((ROWS_PT, D), jnp.float32))
    P2 = P.reshape(NC, N, D)

    out = pl.pallas_call(
        _tc_k2_body,
        grid=(nblk,),
        in_specs=[
            pl.BlockSpec((NC, _BR, D), lambda i: (0, i, 0)),
            pl.BlockSpec((_BR, D), lambda i: (i, 0)),
            pl.BlockSpec((_BR, 1), lambda i: (i, 0)),
        ],
        out_specs=pl.BlockSpec((_BR, D), lambda i: (i, 0)),
        out_shape=jax.ShapeDtypeStruct((N, D), jnp.float32),
    )(P2, g, dinv)
    return out


# speedup vs baseline: 2.2939x; 2.2939x over previous
import functools

import jax
import jax.numpy as jnp
import numpy as np
from jax import lax
from jax.experimental import pallas as pl
from jax.experimental.pallas import tpu as pltpu
from jax.experimental.pallas import tpu_sc as plsc

N = 10000
E = 320000
D = 128
NC = 2
NS = 16
NW = NC * NS
CH = 64
E_PAD = 327680
EPT = E_PAD // NW
NCH = EPT // CH
NPH = 4
PCH = NCH // NPH
NGB = 4
N_PAD = 10240
ROWS_PT = N_PAD // NS
DEG_PT = 640
N_DEG = NS * DEG_PT

PERM = np.array([32 * j + (s // 2 if s % 2 == 0 else 16 + s // 2)
                 for j in range(D // 32) for s in range(32)], dtype=np.int32)

_mesh = plsc.VectorSubcoreMesh(core_axis_name="c", subcore_axis_name="s")


@functools.partial(
    pl.kernel,
    mesh=_mesh,
    out_type=jax.ShapeDtypeStruct((NC, NS, DEG_PT), jnp.float32),
    scratch_types=[
        pltpu.VMEM((NCH // 4, CH), jnp.int32),
        pltpu.VMEM((NCH // 4, CH), jnp.float32),
        pltpu.VMEM_SHARED((N_DEG,), jnp.float32),
        pltpu.SemaphoreType.DMA,
    ],
)
def _sc_deg(dst_hbm, w_hbm, z_hbm, out_hbm, dst_all, w_all, acc, sem):
    cid = lax.axis_index("c")
    sid = lax.axis_index("s")
    wid = sid * NC + cid
    pltpu.sync_copy(z_hbm, acc.at[pl.ds(sid * DEG_PT, DEG_PT)])
    plsc.subcore_barrier()
    qch = NCH // 4

    def body(q, carry):
        for r in range(4):
            t = q * 4 + r
            pltpu.async_copy(w_all.at[t], acc.at[dst_all.at[t]], sem, add=True)
        for r in range(4):
            t = q * 4 + r
            pltpu.make_async_copy(w_all.at[t], acc.at[dst_all.at[t]], sem).wait()
        return carry

    for h in range(4):
        pltpu.sync_copy(dst_hbm.at[wid, pl.ds(h * qch, qch)], dst_all)
        pltpu.sync_copy(w_hbm.at[wid, pl.ds(h * qch, qch)], w_all)
        lax.fori_loop(0, qch // 4, body, 0)
    plsc.subcore_barrier()
    pltpu.sync_copy(acc.at[pl.ds(sid * DEG_PT, DEG_PT)], out_hbm.at[cid, sid])


@functools.partial(
    pl.kernel,
    mesh=_mesh,
    out_type=jax.ShapeDtypeStruct((NC, NS, ROWS_PT, D), jnp.float32),
    scratch_types=[
        pltpu.VMEM((PCH, CH), jnp.int32),
        pltpu.VMEM((PCH, CH), jnp.int32),
        pltpu.VMEM((PCH, CH), jnp.float32),
        pltpu.VMEM((NGB, CH, D), jnp.float32),
        pltpu.VMEM_SHARED((N_PAD, D), jnp.float32),
    ]
    + [pltpu.SemaphoreType.DMA] * (2 * NGB),
)
def _sc_main(hsb_hbm, src_hbm, dst_hbm, w_hbm, z_hbm, out_hbm,
             src_all, dst_all, w_all, rows, acc, *sems):
    sem_g = sems[:NGB]
    sem_s = sems[NGB:]
    cid = lax.axis_index("c")
    sid = lax.axis_index("s")
    wid = sid * NC + cid
    rsl = pl.ds(sid * ROWS_PT, ROWS_PT)
    pltpu.sync_copy(z_hbm, acc.at[rsl])
    plsc.subcore_barrier()

    def _slot(t, r):
        pltpu.make_async_copy(hsb_hbm.at[src_all.at[t]], rows.at[r],
                              sem_g[r]).wait()

        def scale(gg, c2):
            wv16 = w_all[t, pl.ds(gg * 16, 16)]
            for k in range(16):
                s = wv16[k]
                e = gg * 16 + k
                for j in range(D // 16):
                    sl = pl.ds(j * 16, 16)
                    rows[r, e, sl] = rows[r, e, sl] * s
            return c2

        lax.fori_loop(0, CH // 16, scale, 0)
        pltpu.async_copy(rows.at[r], acc.at[dst_all.at[t]], sem_s[r], add=True)
        r2 = (r + 2) % NGB

        @pl.when(t >= 2)
        def _():
            pltpu.make_async_copy(rows.at[r2], acc.at[dst_all.at[t - 2]],
                                  sem_s[r2]).wait()

        @pl.when(t + 2 < PCH)
        def _():
            pltpu.async_copy(hsb_hbm.at[src_all.at[t + 2]], rows.at[r2],
                             sem_g[r2])

    def group(qq, carry):
        for r in range(NGB):
            _slot(qq * NGB + r, r)
        return carry

    for h in range(NPH):
        pltpu.sync_copy(src_hbm.at[wid, pl.ds(h * PCH, PCH)], src_all)
        pltpu.sync_copy(dst_hbm.at[wid, pl.ds(h * PCH, PCH)], dst_all)
        pltpu.sync_copy(w_hbm.at[wid, pl.ds(h * PCH, PCH)], w_all)
        pltpu.async_copy(hsb_hbm.at[src_all.at[0]], rows.at[0], sem_g[0])
        pltpu.async_copy(hsb_hbm.at[src_all.at[1]], rows.at[1], sem_g[1])
        lax.fori_loop(0, PCH // NGB, group, 0)
        pltpu.make_async_copy(rows.at[(PCH - 2) % NGB],
                              acc.at[dst_all.at[PCH - 2]],
                              sem_s[(PCH - 2) % NGB]).wait()
        pltpu.make_async_copy(rows.at[(PCH - 1) % NGB],
                              acc.at[dst_all.at[PCH - 1]],
                              sem_s[(PCH - 1) % NGB]).wait()

    plsc.subcore_barrier()
    pltpu.sync_copy(acc.at[rsl], out_hbm.at[cid, sid])


def _tc_k1_body(text_ref, w_ref, degp_ref, hsb_ref, g_ref, dinv_ref):
    h = jnp.dot(text_ref[...], w_ref[...], preferred_element_type=jnp.float32)
    deg = degp_ref[0] + degp_ref[1] + 2.0
    dinv = lax.rsqrt(deg)
    hsb_ref[...] = h * dinv
    g_ref[...] = (2.0 * dinv * dinv) * h
    dinv_ref[...] = dinv


def _tc_k2_body(p_ref, g_ref, dinv_ref, out_ref):
    s = dinv_ref[...] * (p_ref[0] + p_ref[1]) + g_ref[...]
    out_ref[...] = jnp.where(s >= 0, s, 0.01 * s)


_BR = 1000


def kernel(text, adj_index, adj_weight, W):
    src = adj_index[0]
    dst = adj_index[1]
    pad = E_PAD - E
    zi = jnp.arange(pad, dtype=jnp.int32) % N
    srcp = jnp.concatenate([src, zi]).reshape(NW, NCH, CH)
    dstp = jnp.concatenate([dst, zi]).reshape(NW, NCH, CH)
    wp = jnp.concatenate([adj_weight, jnp.zeros((pad,), jnp.float32)]).reshape(NW, NCH, CH)

    degp = _sc_deg(dstp, wp, jnp.zeros((DEG_PT,), jnp.float32))
    degp3 = degp.reshape(NC, N_DEG, 1)

    nblk = N // _BR
    hsb, g, dinv = pl.pallas_call(
        _tc_k1_body,
        grid=(nblk,),
        in_specs=[
            pl.BlockSpec((_BR, D), lambda i: (i, 0)),
            pl.BlockSpec((D, D), lambda i: (0, 0)),
            pl.BlockSpec((NC, _BR, 1), lambda i: (0, i, 0)),
        ],
        out_specs=[
            pl.BlockSpec((_BR, D), lambda i: (i, 0)),
            pl.BlockSpec((_BR, D), lambda i: (i, 0)),
            pl.BlockSpec((_BR, 1), lambda i: (i, 0)),
        ],
        out_shape=[
            jax.ShapeDtypeStruct((N, D), jnp.float32),
            jax.ShapeDtypeStruct((N, D), jnp.float32),
            jax.ShapeDtypeStruct((N, 1), jnp.float32),
        ],
    )(text, W, degp3)

    P = _sc_main(hsb, srcp, dstp, wp,
                 jnp.zeros((ROWS_PT, D), jnp.float32))
    P2 = P.reshape(NC, N_PAD, D)[:, :N]

    out = pl.pallas_call(
        _tc_k2_body,
        grid=(nblk,),
        in_specs=[
            pl.BlockSpec((NC, _BR, D), lambda i: (0, i, 0)),
            pl.BlockSpec((_BR, D), lambda i: (i, 0)),
            pl.BlockSpec((_BR, 1), lambda i: (i, 0)),
        ],
        out_specs=pl.BlockSpec((_BR, D), lambda i: (i, 0)),
        out_shape=jax.ShapeDtypeStruct((N, D), jnp.float32),
    )(P2, g, dinv)
    return out


# speedup vs baseline: 2.3507x; 1.0247x over previous
import functools

import jax
import jax.numpy as jnp
import numpy as np
from jax import lax
from jax.experimental import pallas as pl
from jax.experimental.pallas import tpu as pltpu
from jax.experimental.pallas import tpu_sc as plsc

N = 10000
E = 320000
D = 128
NC = 2
NS = 16
NW = NC * NS
CH = 128
CHD = 64
E_PAD = 327680
EPT = E_PAD // NW
NCH = EPT // CH
NCHD = EPT // CHD
NPH = 5
PCH = NCH // NPH
NGB = 2
N_PAD = 10240
ROWS_PT = N_PAD // NS
DEG_PT = 640
N_DEG = NS * DEG_PT

PERM = np.array([32 * j + (s // 2 if s % 2 == 0 else 16 + s // 2)
                 for j in range(D // 32) for s in range(32)], dtype=np.int32)

_mesh = plsc.VectorSubcoreMesh(core_axis_name="c", subcore_axis_name="s")


@functools.partial(
    pl.kernel,
    mesh=_mesh,
    out_type=jax.ShapeDtypeStruct((NC, NS, DEG_PT), jnp.float32),
    scratch_types=[
        pltpu.VMEM((NCHD // 4, CHD), jnp.int32),
        pltpu.VMEM((NCHD // 4, CHD), jnp.float32),
        pltpu.VMEM_SHARED((N_DEG,), jnp.float32),
        pltpu.SemaphoreType.DMA,
    ],
)
def _sc_deg(dst_hbm, w_hbm, z_hbm, out_hbm, dst_all, w_all, acc, sem):
    cid = lax.axis_index("c")
    sid = lax.axis_index("s")
    wid = sid * NC + cid
    pltpu.sync_copy(z_hbm, acc.at[pl.ds(sid * DEG_PT, DEG_PT)])
    plsc.subcore_barrier()
    qch = NCHD // 4

    def body(q, carry):
        for r in range(4):
            t = q * 4 + r
            pltpu.async_copy(w_all.at[t], acc.at[dst_all.at[t]], sem, add=True)
        for r in range(4):
            t = q * 4 + r
            pltpu.make_async_copy(w_all.at[t], acc.at[dst_all.at[t]], sem).wait()
        return carry

    for h in range(4):
        pltpu.sync_copy(dst_hbm.at[wid, pl.ds(h * qch, qch)], dst_all)
        pltpu.sync_copy(w_hbm.at[wid, pl.ds(h * qch, qch)], w_all)
        lax.fori_loop(0, qch // 4, body, 0)
    plsc.subcore_barrier()
    pltpu.sync_copy(acc.at[pl.ds(sid * DEG_PT, DEG_PT)], out_hbm.at[cid, sid])


@functools.partial(
    pl.kernel,
    mesh=_mesh,
    out_type=jax.ShapeDtypeStruct((NC, NS, ROWS_PT, D), jnp.float32),
    scratch_types=[
        pltpu.VMEM((PCH, CH), jnp.int32),
        pltpu.VMEM((PCH, CH), jnp.int32),
        pltpu.VMEM((PCH, CH), jnp.float32),
        pltpu.VMEM((NGB, CH, D), jnp.float32),
        pltpu.VMEM_SHARED((N_PAD, D), jnp.float32),
    ]
    + [pltpu.SemaphoreType.DMA] * (2 * NGB),
)
def _sc_main(hsb_hbm, src_hbm, dst_hbm, w_hbm, z_hbm, out_hbm,
             src_all, dst_all, w_all, rows, acc, *sems):
    sem_g = sems[:NGB]
    sem_s = sems[NGB:]
    cid = lax.axis_index("c")
    sid = lax.axis_index("s")
    wid = sid * NC + cid
    rsl = pl.ds(sid * ROWS_PT, ROWS_PT)
    pltpu.sync_copy(z_hbm, acc.at[rsl])
    plsc.subcore_barrier()

    def _slot(t, r):
        pltpu.make_async_copy(hsb_hbm.at[src_all.at[t]], rows.at[r],
                              sem_g[r]).wait()

        def scale(gg, c2):
            wv16 = w_all[t, pl.ds(gg * 16, 16)]
            for k in range(16):
                s = wv16[k]
                e = gg * 16 + k
                for j in range(D // 16):
                    sl = pl.ds(j * 16, 16)
                    rows[r, e, sl] = rows[r, e, sl] * s
            return c2

        lax.fori_loop(0, CH // 16, scale, 0)
        pltpu.async_copy(rows.at[r], acc.at[dst_all.at[t]], sem_s[r], add=True)
        r2 = (r + 2) % NGB

        @pl.when(t >= 2)
        def _():
            pltpu.make_async_copy(rows.at[r2], acc.at[dst_all.at[t - 2]],
                                  sem_s[r2]).wait()

        @pl.when(t + 2 < PCH)
        def _():
            pltpu.async_copy(hsb_hbm.at[src_all.at[t + 2]], rows.at[r2],
                             sem_g[r2])

    def group(qq, carry):
        for r in range(NGB):
            _slot(qq * NGB + r, r)
        return carry

    for h in range(NPH):
        pltpu.sync_copy(src_hbm.at[wid, pl.ds(h * PCH, PCH)], src_all)
        pltpu.sync_copy(dst_hbm.at[wid, pl.ds(h * PCH, PCH)], dst_all)
        pltpu.sync_copy(w_hbm.at[wid, pl.ds(h * PCH, PCH)], w_all)
        pltpu.async_copy(hsb_hbm.at[src_all.at[0]], rows.at[0], sem_g[0])
        pltpu.async_copy(hsb_hbm.at[src_all.at[1]], rows.at[1], sem_g[1])
        lax.fori_loop(0, PCH // NGB, group, 0)
        pltpu.make_async_copy(rows.at[(PCH - 2) % NGB],
                              acc.at[dst_all.at[PCH - 2]],
                              sem_s[(PCH - 2) % NGB]).wait()
        pltpu.make_async_copy(rows.at[(PCH - 1) % NGB],
                              acc.at[dst_all.at[PCH - 1]],
                              sem_s[(PCH - 1) % NGB]).wait()

    plsc.subcore_barrier()
    pltpu.sync_copy(acc.at[rsl], out_hbm.at[cid, sid])


def _tc_k1_body(text_ref, w_ref, degp_ref, hsb_ref, g_ref, dinv_ref):
    h = jnp.dot(text_ref[...], w_ref[...], preferred_element_type=jnp.float32)
    deg = degp_ref[0] + degp_ref[1] + 2.0
    dinv = lax.rsqrt(deg)
    hsb_ref[...] = h * dinv
    g_ref[...] = (2.0 * dinv * dinv) * h
    dinv_ref[...] = dinv


def _tc_k2_body(p_ref, g_ref, dinv_ref, out_ref):
    s = dinv_ref[...] * (p_ref[0] + p_ref[1]) + g_ref[...]
    out_ref[...] = jnp.where(s >= 0, s, 0.01 * s)


_BR = 1000


def kernel(text, adj_index, adj_weight, W):
    src = adj_index[0]
    dst = adj_index[1]
    pad = E_PAD - E
    zi = jnp.arange(pad, dtype=jnp.int32) % N
    srcf = jnp.concatenate([src, zi])
    dstf = jnp.concatenate([dst, zi])
    wf = jnp.concatenate([adj_weight, jnp.zeros((pad,), jnp.float32)])
    srcp = srcf.reshape(NW, NCH, CH)
    dstp = dstf.reshape(NW, NCH, CH)
    wp = wf.reshape(NW, NCH, CH)

    degp = _sc_deg(dstf.reshape(NW, NCHD, CHD), wf.reshape(NW, NCHD, CHD),
                   jnp.zeros((DEG_PT,), jnp.float32))
    degp3 = degp.reshape(NC, N_DEG, 1)

    nblk = N // _BR
    hsb, g, dinv = pl.pallas_call(
        _tc_k1_body,
        grid=(nblk,),
        in_specs=[
            pl.BlockSpec((_BR, D), lambda i: (i, 0)),
            pl.BlockSpec((D, D), lambda i: (0, 0)),
            pl.BlockSpec((NC, _BR, 1), lambda i: (0, i, 0)),
        ],
        out_specs=[
            pl.BlockSpec((_BR, D), lambda i: (i, 0)),
            pl.BlockSpec((_BR, D), lambda i: (i, 0)),
            pl.BlockSpec((_BR, 1), lambda i: (i, 0)),
        ],
        out_shape=[
            jax.ShapeDtypeStruct((N, D), jnp.float32),
            jax.ShapeDtypeStruct((N, D), jnp.float32),
            jax.ShapeDtypeStruct((N, 1), jnp.float32),
        ],
    )(text, W, degp3)

    P = _sc_main(hsb, srcp, dstp, wp,
                 jnp.zeros((ROWS_PT, D), jnp.float32))
    P2 = P.reshape(NC, N_PAD, D)[:, :N]

    out = pl.pallas_call(
        _tc_k2_body,
        grid=(nblk,),
        in_specs=[
            pl.BlockSpec((NC, _BR, D), lambda i: (0, i, 0)),
            pl.BlockSpec((_BR, D), lambda i: (i, 0)),
            pl.BlockSpec((_BR, 1), lambda i: (i, 0)),
        ],
        out_specs=pl.BlockSpec((_BR, D), lambda i: (i, 0)),
        out_shape=jax.ShapeDtypeStruct((N, D), jnp.float32),
    )(P2, g, dinv)
    return out
